# Initial kernel scaffold; baseline (speedup 1.0000x reference)
import jax
import jax.numpy as jnp
from jax.experimental import pallas as pl


def kernel(conv1_w, conv1_b, conv2_w, conv2_b, fc1_w, fc1_b, fc2_w, fc2_b, x):
    raise NotImplementedError("write your pallas kernel here")


# speedup vs baseline: 39.5794x; 39.5794x over previous
import jax
import jax.numpy as jnp
from jax.experimental import pallas as pl
from jax.experimental.pallas import tpu as pltpu


def _fold_conv1(w):
    f32 = jnp.float32
    a = jnp.arange(2)[:, None, None]
    r = jnp.arange(8)[None, :, None]
    kh = jnp.arange(5)[None, None, :]
    rh = (r == 2 * a + kh).astype(f32)
    b = jnp.arange(2)[:, None, None, None]
    pb = jnp.arange(7)[None, :, None, None]
    wc = jnp.arange(32)[None, None, :, None]
    kw = jnp.arange(5)[None, None, None, :]
    cw = (wc == 4 * pb + 2 * b + kw).astype(f32)
    m = jnp.einsum('arh,bpwk,uchk->crwabpu', rh, cw, w.astype(f32))
    m = m.reshape(3, 256, 4, 70)
    m = jnp.pad(m, ((0, 0), (0, 0), (0, 0), (0, 58)))
    return m.reshape(3, 256, 512)


def _fold_conv2(w):
    f32 = jnp.float32
    i = jnp.arange(2)[:, None, None]
    p = jnp.arange(7)[None, :, None]
    r = jnp.arange(5)[None, None, :]
    ai = (p == 2 * i + r).astype(f32)
    m = jnp.einsum('ipr,jqs,ours->pquijo', ai, ai, w.astype(f32))
    m = m.reshape(7, 70, 4, 20)
    m = jnp.pad(m, ((0, 0), (0, 58), (0, 0), (0, 108)))
    return m.reshape(896, 512)


def _pad_lanes(v, n=128):
    return jnp.pad(v, (0, n - v.shape[0])).reshape(1, n)


def _embed(mat, shape):
    out = jnp.zeros(shape, jnp.float32)
    return out.at[:mat.shape[0], :mat.shape[1]].set(mat.astype(jnp.float32))


def _forward_kernel(x_ref, w1_ref, b1_ref, w2_ref, b2_ref,
                    wf1_ref, bf1_ref, wf2_ref, bf2_ref, o_ref):
    blocks = []
    for pa in range(7):
        s = None
        for c in range(3):
            lhs = x_ref[:, 1024 * c + 128 * pa:1024 * c + 128 * pa + 256]
            d = jnp.dot(lhs, w1_ref[c], preferred_element_type=jnp.float32)
            s = d if s is None else s + d
        m = jnp.maximum(jnp.maximum(s[:, 0:128], s[:, 128:256]),
                        jnp.maximum(s[:, 256:384], s[:, 384:512]))
        blocks.append(jnp.maximum(m + b1_ref[...], 0.0))
    act = jnp.concatenate(blocks, axis=1)

    g = jnp.dot(act, w2_ref[...], preferred_element_type=jnp.float32)
    h = jnp.maximum(jnp.maximum(g[:, 0:128], g[:, 128:256]),
                    jnp.maximum(g[:, 256:384], g[:, 384:512]))
    h = jnp.maximum(h + b2_ref[...], 0.0)

    z = jnp.maximum(jnp.dot(h, wf1_ref[...], preferred_element_type=jnp.float32)
                    + bf1_ref[...], 0.0)
    logits = jnp.dot(z, wf2_ref[...],
                     preferred_element_type=jnp.float32) + bf2_ref[...]

    lane = jax.lax.broadcasted_iota(jnp.int32, logits.shape, 1)
    valid = lane < 10
    masked = jnp.where(valid, logits, -jnp.inf)
    mx = jnp.max(masked, axis=-1, keepdims=True)
    e = jnp.where(valid, jnp.exp(logits - mx), 0.0)
    lse = jnp.log(jnp.sum(e, axis=-1, keepdims=True))
    o_ref[...] = jnp.where(valid, logits - mx - lse, 0.0)


def kernel(conv1_w, conv1_b, conv2_w, conv2_b, fc1_w, fc1_b, fc2_w, fc2_b, x,
           tb=256):
    B = x.shape[0]
    xf = x.astype(jnp.float32).reshape(B, 3 * 32 * 32)
    tb = min(tb, max(8, B))
    Bp = ((B + tb - 1) // tb) * tb
    if Bp != B:
        xf = jnp.pad(xf, ((0, Bp - B), (0, 0)))

    w1 = _fold_conv1(conv1_w)
    b1t = _pad_lanes(jnp.tile(conv1_b.astype(jnp.float32), 7))
    w2 = _fold_conv2(conv2_w)
    b2p = _pad_lanes(conv2_b.astype(jnp.float32))
    wf1 = _embed(fc1_w.T, (128, 128))
    bf1 = _pad_lanes(fc1_b.astype(jnp.float32))
    wf2 = _embed(fc2_w.T, (128, 128))
    bf2 = _pad_lanes(fc2_b.astype(jnp.float32))

    out = pl.pallas_call(
        _forward_kernel,
        out_shape=jax.ShapeDtypeStruct((Bp, 128), jnp.float32),
        grid=(Bp // tb,),
        in_specs=[
            pl.BlockSpec((tb, 3072), lambda i: (i, 0)),
            pl.BlockSpec((3, 256, 512), lambda i: (0, 0, 0)),
            pl.BlockSpec((1, 128), lambda i: (0, 0)),
            pl.BlockSpec((896, 512), lambda i: (0, 0)),
            pl.BlockSpec((1, 128), lambda i: (0, 0)),
            pl.BlockSpec((128, 128), lambda i: (0, 0)),
            pl.BlockSpec((1, 128), lambda i: (0, 0)),
            pl.BlockSpec((128, 128), lambda i: (0, 0)),
            pl.BlockSpec((1, 128), lambda i: (0, 0)),
        ],
        out_specs=pl.BlockSpec((tb, 128), lambda i: (i, 0)),
        compiler_params=pltpu.CompilerParams(
            dimension_semantics=("parallel",),
            vmem_limit_bytes=64 * 1024 * 1024),
    )(xf, w1, b1t, w2, b2p, wf1, bf1, wf2, bf2)
    return out[:B, :10]
